# baseline (speedup 1.0000x reference)
import functools

import jax
import jax.numpy as jnp
from jax import lax
from jax.experimental import pallas as pl
from jax.experimental.pallas import tpu as pltpu
from jax.experimental.pallas import tpu_sc as plsc

VOCAB = 100000
EMB = 128
HID = 1024
MAXPOS = 4096
TYPES = 2
EPS = 1e-12

NC = 2
NS = 16
NW = NC * NS

GATHER_CHUNK = 128


def _sc_gather(table, input_ids, n_tokens):
    bsz, seq_len = input_ids.shape
    rows_per_w = n_tokens // NW
    n_chunks = rows_per_w // GATHER_CHUNK
    w_per_b = NW // bsz
    mesh = plsc.VectorSubcoreMesh(core_axis_name="c", subcore_axis_name="s")

    @functools.partial(
        pl.kernel,
        out_type=jax.ShapeDtypeStruct((n_tokens, EMB), jnp.float32),
        mesh=mesh,
        scratch_types=[
            pltpu.VMEM((n_chunks, GATHER_CHUNK), jnp.int32),
            pltpu.VMEM((rows_per_w, EMB), jnp.float32),
            pltpu.SemaphoreType.DMA,
            pltpu.SemaphoreType.DMA,
        ],
    )
    def gather_kernel(table_hbm, idx_hbm, out_hbm, idx_v, rows_v, gsem, wsem):
        wid = lax.axis_index("s") * NC + lax.axis_index("c")
        b = wid // w_per_b
        col0 = (wid % w_per_b) * rows_per_w
        for j in range(n_chunks):
            pltpu.sync_copy(
                idx_hbm.at[b, pl.ds(col0 + j * GATHER_CHUNK, GATHER_CHUNK)],
                idx_v.at[j],
            )
        gathers = []
        for j in range(n_chunks):
            gathers.append(
                pltpu.async_copy(
                    table_hbm.at[idx_v.at[j]],
                    rows_v.at[pl.ds(j * GATHER_CHUNK, GATHER_CHUNK)],
                    gsem,
                )
            )
        writes = []
        for j in range(n_chunks):
            gathers[j].wait()
            writes.append(
                pltpu.async_copy(
                    rows_v.at[pl.ds(j * GATHER_CHUNK, GATHER_CHUNK)],
                    out_hbm.at[pl.ds(wid * rows_per_w + j * GATHER_CHUNK,
                                     GATHER_CHUNK)],
                    wsem,
                )
            )
        for c in writes:
            c.wait()

    return gather_kernel(table, input_ids)


def _tc_body(x_ref, pos_ref, tt_ref, seg_ref, gamma_ref, beta_ref, w_ref,
             b_ref, o_ref):
    x = x_ref[...]
    pos = pos_ref[...]
    ttf = tt_ref[0].astype(jnp.float32)
    seg = seg_ref[...]
    d = (seg[1] - seg[0])[None, :]
    seg_term = lax.dot_general(
        ttf, d, (((0,), (0,)), ((), ())), preferred_element_type=jnp.float32)
    e = x + pos + seg[0][None, :] + seg_term
    inv = 1.0 / EMB
    s1 = jnp.sum(e, axis=1, keepdims=True)
    s2 = jnp.sum(e * e, axis=1, keepdims=True)
    m = s1 * inv
    var = s2 * inv - m * m
    y = (e - m) * lax.rsqrt(var + EPS) * gamma_ref[...] + beta_ref[...]
    o_ref[...] = lax.dot_general(
        y.astype(jnp.bfloat16), w_ref[...], (((1,), (1,)), ((), ())),
        preferred_element_type=jnp.float32,
    ) + b_ref[...]


def _tc_body_alias(x_ref, pos_ref, tt_ref, seg_ref, gamma_ref, beta_ref,
                   w_ref, b_ref, prev_ref, o_ref):
    del prev_ref
    _tc_body(x_ref, pos_ref, tt_ref, seg_ref, gamma_ref, beta_ref, w_ref,
             b_ref, o_ref)


def _tc_fused(gathered, pos_table, tt_half, seg_table, gamma2, beta2, W_proj,
              b2, n_tokens, seq_len, b0, prev=None):
    tok_blk = 2048
    n_half = tt_half.size
    n_batch = n_half // seq_len
    blocks_per_seq = seq_len // tok_blk
    tt3 = tt_half.reshape(n_half // tok_blk, 1, tok_blk)
    in_specs = [
        pl.BlockSpec((tok_blk, EMB),
                     lambda s, b: (b * blocks_per_seq + s, 0)),
        pl.BlockSpec((tok_blk, EMB), lambda s, b: (s, 0)),
        pl.BlockSpec((1, 1, tok_blk),
                     lambda s, b: (b * blocks_per_seq + s, 0, 0)),
        pl.BlockSpec((TYPES, EMB), lambda s, b: (0, 0)),
        pl.BlockSpec((1, EMB), lambda s, b: (0, 0)),
        pl.BlockSpec((1, EMB), lambda s, b: (0, 0)),
        pl.BlockSpec((HID, EMB), lambda s, b: (0, 0)),
        pl.BlockSpec((1, HID), lambda s, b: (0, 0)),
    ]
    args = [gathered, pos_table, tt3, seg_table, gamma2, beta2, W_proj, b2]
    body = _tc_body
    aliases = {}
    if prev is not None:
        in_specs.append(pl.BlockSpec((8, 128), lambda s, b: (0, 0)))
        args.append(prev)
        body = _tc_body_alias
        aliases = {8: 0}
    blk0 = b0 * blocks_per_seq
    return pl.pallas_call(
        body,
        grid=(blocks_per_seq, n_batch),
        in_specs=in_specs,
        out_specs=pl.BlockSpec((tok_blk, HID),
                               lambda s, b: (blk0 + b * blocks_per_seq + s, 0)),
        out_shape=jax.ShapeDtypeStruct((n_tokens, HID), jnp.float32),
        input_output_aliases=aliases,
    )(*args)


def kernel(input_ids, token_type_ids, tok_table, pos_table, seg_table,
           ln_gamma, ln_beta, W_proj, b_proj):
    bsz, seq_len = input_ids.shape
    n_tokens = bsz * seq_len
    half_b = bsz // 2
    ids = input_ids.astype(jnp.int32)
    tt = token_type_ids.astype(jnp.int32)
    g1 = _sc_gather(tok_table, ids[:half_b], n_tokens // 2)
    g2 = _sc_gather(tok_table, ids[half_b:], n_tokens // 2)
    gamma2 = ln_gamma.reshape(1, EMB)
    beta2 = ln_beta.reshape(1, EMB)
    Wb = W_proj.astype(jnp.bfloat16)
    b2 = b_proj.reshape(1, HID)
    o1 = _tc_fused(g1, pos_table, tt[:half_b], seg_table, gamma2, beta2,
                   Wb, b2, n_tokens, seq_len, b0=0)
    o2 = _tc_fused(g2, pos_table, tt[half_b:], seg_table, gamma2, beta2,
                   Wb, b2, n_tokens, seq_len, b0=half_b, prev=o1)
    return o2.reshape(bsz, seq_len, HID)


# speedup vs baseline: 1.0320x; 1.0320x over previous
import functools

import jax
import jax.numpy as jnp
from jax import lax
from jax.experimental import pallas as pl
from jax.experimental.pallas import tpu as pltpu
from jax.experimental.pallas import tpu_sc as plsc

VOCAB = 100000
EMB = 128
HID = 1024
MAXPOS = 4096
TYPES = 2
EPS = 1e-12

NC = 2
NS = 16
NW = NC * NS

GATHER_CHUNK = 128


def _sc_gather(table, input_ids, n_tokens):
    bsz, seq_len = input_ids.shape
    rows_per_w = n_tokens // NW
    n_chunks = rows_per_w // GATHER_CHUNK
    w_per_b = NW // bsz
    mesh = plsc.VectorSubcoreMesh(core_axis_name="c", subcore_axis_name="s")

    @functools.partial(
        pl.kernel,
        out_type=jax.ShapeDtypeStruct((n_tokens, EMB), jnp.float32),
        mesh=mesh,
        scratch_types=[
            pltpu.VMEM((n_chunks, GATHER_CHUNK), jnp.int32),
            pltpu.VMEM((rows_per_w, EMB), jnp.float32),
            pltpu.SemaphoreType.DMA,
            pltpu.SemaphoreType.DMA,
        ],
    )
    def gather_kernel(table_hbm, idx_hbm, out_hbm, idx_v, rows_v, gsem, wsem):
        wid = lax.axis_index("s") * NC + lax.axis_index("c")
        b = wid // w_per_b
        col0 = (wid % w_per_b) * rows_per_w
        for j in range(n_chunks):
            pltpu.sync_copy(
                idx_hbm.at[b, pl.ds(col0 + j * GATHER_CHUNK, GATHER_CHUNK)],
                idx_v.at[j],
            )
        gathers = []
        for j in range(n_chunks):
            gathers.append(
                pltpu.async_copy(
                    table_hbm.at[idx_v.at[j]],
                    rows_v.at[pl.ds(j * GATHER_CHUNK, GATHER_CHUNK)],
                    gsem,
                )
            )
        writes = []
        for j in range(n_chunks):
            gathers[j].wait()
            writes.append(
                pltpu.async_copy(
                    rows_v.at[pl.ds(j * GATHER_CHUNK, GATHER_CHUNK)],
                    out_hbm.at[pl.ds(wid * rows_per_w + j * GATHER_CHUNK,
                                     GATHER_CHUNK)],
                    wsem,
                )
            )
        for c in writes:
            c.wait()

    return gather_kernel(table, input_ids)


def _tc_body(x_ref, pos_ref, tt_ref, seg_ref, gamma_ref, beta_ref, w_ref,
             b_ref, o_ref):
    x = x_ref[...]
    pos = pos_ref[...]
    ttf = tt_ref[0].astype(jnp.float32)
    seg = seg_ref[...]
    d = (seg[1] - seg[0])[None, :]
    seg_term = lax.dot_general(
        ttf, d, (((0,), (0,)), ((), ())), preferred_element_type=jnp.float32)
    e = x + pos + seg[0][None, :] + seg_term
    inv = 1.0 / EMB
    s1 = jnp.sum(e, axis=1, keepdims=True)
    s2 = jnp.sum(e * e, axis=1, keepdims=True)
    m = s1 * inv
    var = s2 * inv - m * m
    y = (e - m) * lax.rsqrt(var + EPS) * gamma_ref[...] + beta_ref[...]
    o_ref[...] = lax.dot_general(
        y.astype(jnp.bfloat16), w_ref[...], (((1,), (1,)), ((), ())),
        preferred_element_type=jnp.float32,
    ) + b_ref[...]


def _tc_fused(gathered, pos_table, tt_ids, seg_table, gamma2, beta2, W_proj,
              b2, n_tokens, seq_len):
    tok_blk = 4096
    n_batch = n_tokens // seq_len
    blocks_per_seq = seq_len // tok_blk
    tt3 = tt_ids.reshape(n_tokens // tok_blk, 1, tok_blk)
    return pl.pallas_call(
        _tc_body,
        grid=(blocks_per_seq, n_batch),
        in_specs=[
            pl.BlockSpec((tok_blk, EMB),
                         lambda s, b: (b * blocks_per_seq + s, 0)),
            pl.BlockSpec((tok_blk, EMB), lambda s, b: (s, 0)),
            pl.BlockSpec((1, 1, tok_blk),
                         lambda s, b: (b * blocks_per_seq + s, 0, 0)),
            pl.BlockSpec((TYPES, EMB), lambda s, b: (0, 0)),
            pl.BlockSpec((1, EMB), lambda s, b: (0, 0)),
            pl.BlockSpec((1, EMB), lambda s, b: (0, 0)),
            pl.BlockSpec((HID, EMB), lambda s, b: (0, 0)),
            pl.BlockSpec((1, HID), lambda s, b: (0, 0)),
        ],
        out_specs=pl.BlockSpec((tok_blk, HID),
                               lambda s, b: (b * blocks_per_seq + s, 0)),
        out_shape=jax.ShapeDtypeStruct((n_tokens, HID), jnp.float32),
    )(gathered, pos_table, tt3, seg_table, gamma2, beta2, W_proj, b2)


def kernel(input_ids, token_type_ids, tok_table, pos_table, seg_table,
           ln_gamma, ln_beta, W_proj, b_proj):
    bsz, seq_len = input_ids.shape
    n_tokens = bsz * seq_len
    gathered = _sc_gather(tok_table, input_ids.astype(jnp.int32), n_tokens)
    out = _tc_fused(
        gathered, pos_table, token_type_ids.astype(jnp.int32), seg_table,
        ln_gamma.reshape(1, EMB), ln_beta.reshape(1, EMB),
        W_proj.astype(jnp.bfloat16), b_proj.reshape(1, HID),
        n_tokens, seq_len,
    )
    return out.reshape(bsz, seq_len, HID)


# speedup vs baseline: 1.0510x; 1.0184x over previous
import functools

import jax
import jax.numpy as jnp
from jax import lax
from jax.experimental import pallas as pl
from jax.experimental.pallas import tpu as pltpu
from jax.experimental.pallas import tpu_sc as plsc

VOCAB = 100000
EMB = 128
HID = 1024
MAXPOS = 4096
TYPES = 2
EPS = 1e-12

NC = 2
NS = 16
NW = NC * NS

GATHER_CHUNK = 128


def _sc_gather(table, input_ids, n_tokens):
    bsz, seq_len = input_ids.shape
    rows_per_w = n_tokens // NW
    n_chunks = rows_per_w // GATHER_CHUNK
    w_per_b = NW // bsz
    mesh = plsc.VectorSubcoreMesh(core_axis_name="c", subcore_axis_name="s")

    @functools.partial(
        pl.kernel,
        out_type=jax.ShapeDtypeStruct((n_tokens, EMB), jnp.float32),
        mesh=mesh,
        scratch_types=[
            pltpu.VMEM((n_chunks, GATHER_CHUNK), jnp.int32),
            pltpu.VMEM((rows_per_w, EMB), jnp.float32),
            pltpu.SemaphoreType.DMA,
            pltpu.SemaphoreType.DMA,
        ],
    )
    def gather_kernel(table_hbm, idx_hbm, out_hbm, idx_v, rows_v, gsem, wsem):
        wid = lax.axis_index("s") * NC + lax.axis_index("c")
        b = wid // w_per_b
        col0 = (wid % w_per_b) * rows_per_w
        for j in range(n_chunks):
            pltpu.sync_copy(
                idx_hbm.at[b, pl.ds(col0 + j * GATHER_CHUNK, GATHER_CHUNK)],
                idx_v.at[j],
            )
        gathers = []
        for j in range(n_chunks):
            gathers.append(
                pltpu.async_copy(
                    table_hbm.at[idx_v.at[j]],
                    rows_v.at[pl.ds(j * GATHER_CHUNK, GATHER_CHUNK)],
                    gsem,
                )
            )
        writes = []
        for j in range(n_chunks):
            gathers[j].wait()
            writes.append(
                pltpu.async_copy(
                    rows_v.at[pl.ds(j * GATHER_CHUNK, GATHER_CHUNK)],
                    out_hbm.at[pl.ds(wid * rows_per_w + j * GATHER_CHUNK,
                                     GATHER_CHUNK)],
                    wsem,
                )
            )
        for c in writes:
            c.wait()

    return gather_kernel(table, input_ids)


def _tc_body(x_ref, pos_ref, tt_ref, seg_ref, gamma_ref, beta_ref, w_ref,
             b_ref, o_ref):
    x = x_ref[...]
    pos = pos_ref[...]
    ttf = tt_ref[0].astype(jnp.float32)
    seg = seg_ref[...]
    d = (seg[1] - seg[0])[None, :]
    seg_term = lax.dot_general(
        ttf, d, (((0,), (0,)), ((), ())), preferred_element_type=jnp.float32)
    e = x + pos + seg[0][None, :] + seg_term
    inv = 1.0 / EMB
    s1 = jnp.sum(e, axis=1, keepdims=True)
    s2 = jnp.sum(e * e, axis=1, keepdims=True)
    m = s1 * inv
    var = s2 * inv - m * m
    y = (e - m) * lax.rsqrt(var + EPS) * gamma_ref[...] + beta_ref[...]
    o_ref[...] = lax.dot_general(
        y.astype(jnp.bfloat16), w_ref[...], (((1,), (1,)), ((), ())),
        preferred_element_type=jnp.float32,
    ) + b_ref[...]


def _tc_fused(gathered, pos_table, tt_ids, seg_table, gamma2, beta2, W_proj,
              b2, n_tokens, seq_len):
    tok_blk = 2048
    n_batch = n_tokens // seq_len
    blocks_per_seq = seq_len // tok_blk
    tt3 = tt_ids.reshape(n_tokens // tok_blk, 1, tok_blk)
    return pl.pallas_call(
        _tc_body,
        grid=(blocks_per_seq, n_batch),
        in_specs=[
            pl.BlockSpec((tok_blk, EMB),
                         lambda s, b: (b * blocks_per_seq + s, 0)),
            pl.BlockSpec((tok_blk, EMB), lambda s, b: (s, 0)),
            pl.BlockSpec((1, 1, tok_blk),
                         lambda s, b: (b * blocks_per_seq + s, 0, 0)),
            pl.BlockSpec((TYPES, EMB), lambda s, b: (0, 0)),
            pl.BlockSpec((1, EMB), lambda s, b: (0, 0)),
            pl.BlockSpec((1, EMB), lambda s, b: (0, 0)),
            pl.BlockSpec((HID, EMB), lambda s, b: (0, 0)),
            pl.BlockSpec((1, HID), lambda s, b: (0, 0)),
        ],
        out_specs=pl.BlockSpec((tok_blk, HID),
                               lambda s, b: (b * blocks_per_seq + s, 0)),
        out_shape=jax.ShapeDtypeStruct((n_tokens, HID), jnp.float32),
    )(gathered, pos_table, tt3, seg_table, gamma2, beta2, W_proj, b2)


def kernel(input_ids, token_type_ids, tok_table, pos_table, seg_table,
           ln_gamma, ln_beta, W_proj, b_proj):
    bsz, seq_len = input_ids.shape
    n_tokens = bsz * seq_len
    gathered = _sc_gather(tok_table, input_ids.astype(jnp.int32), n_tokens)
    out = _tc_fused(
        gathered, pos_table, token_type_ids.astype(jnp.int32), seg_table,
        ln_gamma.reshape(1, EMB), ln_beta.reshape(1, EMB),
        W_proj.astype(jnp.bfloat16), b_proj.reshape(1, HID),
        n_tokens, seq_len,
    )
    return out.reshape(bsz, seq_len, HID)
